# baseline (speedup 1.0000x reference)
import dataclasses
import functools

import jax
import jax.numpy as jnp
from jax import lax
from jax.experimental import pallas as pl
from jax.experimental.pallas import tpu as pltpu
from jax.experimental.pallas import tpu_sc as plsc

N = 10000
E = 320000
D = 128
H = 16
C = 10

NC, NS = 2, 16
NW = NC * NS
L = 16

EDGES_A = 9984
TAIL = E - NW * EDGES_A
TAIL_BASE = NW * EDGES_A
GRP_T = TAIL // L

FG = 4
FPT = H // FG
NSH = NW // FG
EDGES_SH = 39936
CHUNK = 1536
NCHUNK = EDGES_SH // CHUNK
GRP = CHUNK // L

_mesh = plsc.VectorSubcoreMesh(core_axis_name="c", subcore_axis_name="s")
_cp = pltpu.CompilerParams()
if "needs_layout_passes" in pltpu.CompilerParams.__dataclass_fields__:
    _cp = dataclasses.replace(_cp, needs_layout_passes=False)


@functools.partial(
    pl.kernel,
    out_type=jax.ShapeDtypeStruct((NW, N), jnp.float32),
    mesh=_mesh,
    scratch_types=[
        pltpu.VMEM((2, EDGES_A), jnp.int32),
        pltpu.VMEM((2, TAIL), jnp.int32),
        pltpu.VMEM((N,), jnp.float32),
    ],
    compiler_params=_cp,
)
def _sc_degree(edge_hbm, deg_hbm, ei_v, tail_v, deg_v):
    wid = lax.axis_index("s") * NC + lax.axis_index("c")
    pltpu.sync_copy(
        edge_hbm.at[pl.ds(0, 2), pl.ds(wid * EDGES_A, EDGES_A)], ei_v)

    zeros = jnp.zeros((L,), jnp.float32)

    @pl.loop(0, N // L, unroll=8)
    def _zero(i):
        deg_v[pl.ds(i * L, L)] = zeros

    ones = jnp.ones((L,), jnp.float32)

    @plsc.parallel_loop(0, EDGES_A // L, unroll=8)
    def _hist(i):
        idx = ei_v[1, pl.ds(i * L, L)]
        plsc.addupdate_scatter(deg_v, [idx], ones)

    @pl.when(wid == 0)
    def _tail():
        pltpu.sync_copy(
            edge_hbm.at[pl.ds(0, 2), pl.ds(TAIL_BASE, TAIL)], tail_v)

        @pl.loop(0, GRP_T)
        def _hist_t(i):
            idx = tail_v[1, pl.ds(i * L, L)]
            plsc.addupdate_scatter(deg_v, [idx], ones)

    pltpu.sync_copy(deg_v, deg_hbm.at[wid])


def _tc_matmul_body(x_ref, w1_ref, xw_ref):
    xw_ref[...] = jnp.dot(x_ref[...], w1_ref[...],
                          preferred_element_type=jnp.float32)


_tc_matmul = pl.pallas_call(
    _tc_matmul_body,
    out_shape=jax.ShapeDtypeStruct((N, H), jnp.float32),
)


def _tc_scale_body(degp_ref, xw_ref, dis_ref, xw2t_ref, xwp_ref):
    deg = jnp.sum(degp_ref[...], axis=0, keepdims=True) + 1.0
    dis = lax.rsqrt(deg)
    dis_ref[...] = dis
    xw2t = dis * xw_ref[...].T
    xw2t_ref[...] = xw2t
    lo = lax.bitcast_convert_type(
        lax.convert_element_type(xw2t[0:H // 2], jnp.bfloat16), jnp.uint16)
    hi = lax.bitcast_convert_type(
        lax.convert_element_type(xw2t[H // 2:], jnp.bfloat16), jnp.uint16)
    packed = (hi.astype(jnp.uint32) << 16) | lo.astype(jnp.uint32)
    xwp_ref[...] = lax.bitcast_convert_type(packed, jnp.int32)


_tc_scale = pl.pallas_call(
    _tc_scale_body,
    out_shape=[
        jax.ShapeDtypeStruct((1, N), jnp.float32),
        jax.ShapeDtypeStruct((H, N), jnp.float32),
        jax.ShapeDtypeStruct((H // 2, N), jnp.int32),
    ],
)


@functools.partial(
    pl.kernel,
    out_type=[
        jax.ShapeDtypeStruct((NSH, H, N), jnp.float32),
        jax.ShapeDtypeStruct((NW, N), jnp.float32),
    ],
    mesh=_mesh,
    scratch_types=[
        pltpu.VMEM((N,), jnp.float32),
        *[pltpu.VMEM((N,), jnp.int32) for _ in range(FPT // 2)],
        *[pltpu.VMEM((N,), jnp.float32) for _ in range(FPT)],
        pltpu.VMEM((N,), jnp.float32),
        pltpu.VMEM((2, CHUNK), jnp.int32),
        pltpu.VMEM((2, CHUNK), jnp.int32),
        pltpu.VMEM((2, TAIL), jnp.int32),
        pltpu.SemaphoreType.DMA,
        pltpu.SemaphoreType.DMA,
    ],
    compiler_params=_cp,
)
def _sc_edges(edge_hbm, dis_hbm, xwp_hbm, out_hbm, s_hbm,
              dis_v, xwp0, xwp1, ac0, ac1, ac2, ac3,
              s_v, eiA, eiB, tail_v, semA, semB):
    xwp_cols = (xwp0, xwp1)
    acc_cols = (ac0, ac1, ac2, ac3)
    wid = lax.axis_index("s") * NC + lax.axis_index("c")
    fg = wid % FG
    sh = wid // FG
    ebase = sh * EDGES_SH

    def start(c, buf, sem):
        pltpu.async_copy(
            edge_hbm.at[pl.ds(0, 2), pl.ds(ebase + c * CHUNK, CHUNK)], buf, sem)

    def drain(buf, sem):
        pltpu.make_async_copy(
            edge_hbm.at[pl.ds(0, 2), pl.ds(0, CHUNK)], buf, sem).wait()

    start(0, eiA, semA)

    pltpu.sync_copy(dis_hbm, dis_v)
    for j in range(FPT // 2):
        pltpu.sync_copy(xwp_hbm.at[fg * 2 + j], xwp_cols[j])

    zeros = jnp.zeros((L,), jnp.float32)

    @pl.loop(0, N // L, unroll=8)
    def _zero(i):
        s_v[pl.ds(i * L, L)] = zeros
        for f in range(FPT):
            acc_cols[f][pl.ds(i * L, L)] = zeros

    mask_s = jnp.broadcast_to(fg == 0, (L,))

    def process(buf, ngrp=GRP):
        @plsc.parallel_loop(0, ngrp, unroll=6)
        def _grp(i):
            s16 = buf[0, pl.ds(i * L, L)]
            d16 = buf[1, pl.ds(i * L, L)]
            b = plsc.load_gather(dis_v, [d16])
            for j in range(FPT // 2):
                gp = plsc.load_gather(xwp_cols[j], [s16])
                g0, g1 = plsc.unpack(
                    plsc.bitcast(gp, jnp.bfloat16),
                    format=plsc.PackFormat.INTERLEAVED,
                    preferred_element_type=jnp.float32)
                plsc.addupdate_scatter(acc_cols[2 * j], [d16], g0 * b)
                plsc.addupdate_scatter(acc_cols[2 * j + 1], [d16], g1 * b)
            plsc.addupdate_scatter(s_v, [s16], b, mask=mask_s)

    @pl.loop(0, NCHUNK, step=2)
    def _chunk(c):
        drain(eiA, semA)
        start(c + 1, eiB, semB)
        process(eiA)
        drain(eiB, semB)

        @pl.when(c + 2 < NCHUNK)
        def _():
            start(c + 2, eiA, semA)

        process(eiB)

    @pl.when(sh == 0)
    def _tail():
        pltpu.sync_copy(
            edge_hbm.at[pl.ds(0, 2), pl.ds(TAIL_BASE, TAIL)], tail_v)
        process(tail_v, GRP_T)

    for j in range(FPT // 2):
        pltpu.sync_copy(acc_cols[2 * j], out_hbm.at[sh, fg * 2 + j])
        pltpu.sync_copy(acc_cols[2 * j + 1], out_hbm.at[sh, fg * 2 + j + H // 2])
    pltpu.sync_copy(s_v, s_hbm.at[wid])


def _tc_finish_body(outp_ref, sp_ref, dis_ref, xw2t_ref, b1_ref, w2_ref,
                    b2_ref, o_ref):
    dis = dis_ref[...]
    edge = jnp.sum(outp_ref[...], axis=0)
    h = edge + dis * xw2t_ref[...] + b1_ref[...]
    h = jnp.maximum(h, 0.0)
    s = jnp.sum(sp_ref[...], axis=0, keepdims=True)
    cvec = dis * (s + dis)
    v = jnp.sum(h * cvec, axis=1, keepdims=True)
    pooled = jnp.sum(v * w2_ref[...], axis=0, keepdims=True) / N + b2_ref[...]
    m = jnp.max(pooled)
    e = jnp.exp(pooled - m)
    o_ref[...] = e / jnp.sum(e)


_tc_finish = pl.pallas_call(
    _tc_finish_body,
    out_shape=jax.ShapeDtypeStruct((1, C), jnp.float32),
)


def kernel(x, edge_index, W1, b1, W2, b2):
    edge_index = edge_index.astype(jnp.int32)

    xw = _tc_matmul(x, W1)
    deg_part = _sc_degree(edge_index)
    dis2d, xw2t, xwp = _tc_scale(deg_part, xw)
    dis = dis2d.reshape(N)

    out_part, s_part = _sc_edges(edge_index, dis, xwp)
    out = _tc_finish(out_part, s_part, dis2d, xw2t,
                     b1.reshape(H, 1), W2, b2.reshape(1, C))
    return out


# speedup vs baseline: 1.0456x; 1.0456x over previous
import dataclasses
import functools

import jax
import jax.numpy as jnp
from jax import lax
from jax.experimental import pallas as pl
from jax.experimental.pallas import tpu as pltpu
from jax.experimental.pallas import tpu_sc as plsc

N = 10000
E = 320000
D = 128
H = 16
C = 10

NC, NS = 2, 16
NW = NC * NS
L = 16

EDGES_A = 9984
TAIL = E - NW * EDGES_A
TAIL_BASE = NW * EDGES_A
GRP_T = TAIL // L

FG = 4
FPT = H // FG
NSH = NW // FG
EDGES_SH = 39936
CHUNK = 1536
NCHUNK = EDGES_SH // CHUNK
GRP = CHUNK // L

_mesh = plsc.VectorSubcoreMesh(core_axis_name="c", subcore_axis_name="s")
_cp = pltpu.CompilerParams()
if "needs_layout_passes" in pltpu.CompilerParams.__dataclass_fields__:
    _cp = dataclasses.replace(_cp, needs_layout_passes=False)


@functools.partial(
    pl.kernel,
    out_type=jax.ShapeDtypeStruct((NW, N), jnp.float32),
    mesh=_mesh,
    scratch_types=[
        pltpu.VMEM((2, EDGES_A), jnp.int32),
        pltpu.VMEM((2, TAIL), jnp.int32),
        pltpu.VMEM((N,), jnp.float32),
        pltpu.SemaphoreType.DMA,
    ],
    compiler_params=_cp,
)
def _sc_degree(edge_hbm, deg_hbm, ei_v, tail_v, deg_v, sem):
    wid = lax.axis_index("s") * NC + lax.axis_index("c")
    cp = pltpu.make_async_copy(
        edge_hbm.at[pl.ds(0, 2), pl.ds(wid * EDGES_A, EDGES_A)], ei_v, sem)
    cp.start()

    zeros = jnp.zeros((L,), jnp.float32)

    @pl.loop(0, N // L, unroll=8)
    def _zero(i):
        deg_v[pl.ds(i * L, L)] = zeros

    cp.wait()

    ones = jnp.ones((L,), jnp.float32)

    @plsc.parallel_loop(0, EDGES_A // L, unroll=8)
    def _hist(i):
        idx = ei_v[1, pl.ds(i * L, L)]
        plsc.addupdate_scatter(deg_v, [idx], ones)

    @pl.when(wid == 0)
    def _tail():
        pltpu.sync_copy(
            edge_hbm.at[pl.ds(0, 2), pl.ds(TAIL_BASE, TAIL)], tail_v)

        @pl.loop(0, GRP_T)
        def _hist_t(i):
            idx = tail_v[1, pl.ds(i * L, L)]
            plsc.addupdate_scatter(deg_v, [idx], ones)

    pltpu.sync_copy(deg_v, deg_hbm.at[wid])


def _tc_matmul_body(x_ref, w1_ref, xw_ref):
    xw_ref[...] = jnp.dot(x_ref[...], w1_ref[...],
                          preferred_element_type=jnp.float32)


_tc_matmul = pl.pallas_call(
    _tc_matmul_body,
    out_shape=jax.ShapeDtypeStruct((N, H), jnp.float32),
)


def _tc_scale_body(degp_ref, xw_ref, dis_ref, xw2t_ref, xwp_ref):
    deg = jnp.sum(degp_ref[...], axis=0, keepdims=True) + 1.0
    dis = lax.rsqrt(deg)
    dis_ref[...] = dis
    xw2t = dis * xw_ref[...].T
    xw2t_ref[...] = xw2t
    lo = lax.bitcast_convert_type(
        lax.convert_element_type(xw2t[0:H // 2], jnp.bfloat16), jnp.uint16)
    hi = lax.bitcast_convert_type(
        lax.convert_element_type(xw2t[H // 2:], jnp.bfloat16), jnp.uint16)
    packed = (hi.astype(jnp.uint32) << 16) | lo.astype(jnp.uint32)
    xwp_ref[...] = lax.bitcast_convert_type(packed, jnp.int32)


_tc_scale = pl.pallas_call(
    _tc_scale_body,
    out_shape=[
        jax.ShapeDtypeStruct((1, N), jnp.float32),
        jax.ShapeDtypeStruct((H, N), jnp.float32),
        jax.ShapeDtypeStruct((H // 2, N), jnp.int32),
    ],
)


@functools.partial(
    pl.kernel,
    out_type=[
        jax.ShapeDtypeStruct((NSH, H, N), jnp.float32),
        jax.ShapeDtypeStruct((NW, N), jnp.float32),
    ],
    mesh=_mesh,
    scratch_types=[
        pltpu.VMEM((N,), jnp.float32),
        *[pltpu.VMEM((N,), jnp.int32) for _ in range(FPT // 2)],
        *[pltpu.VMEM((N,), jnp.float32) for _ in range(FPT)],
        pltpu.VMEM((N,), jnp.float32),
        pltpu.VMEM((2, CHUNK), jnp.int32),
        pltpu.VMEM((2, CHUNK), jnp.int32),
        pltpu.VMEM((2, TAIL), jnp.int32),
        pltpu.SemaphoreType.DMA,
        pltpu.SemaphoreType.DMA,
        pltpu.SemaphoreType.DMA,
    ],
    compiler_params=_cp,
)
def _sc_edges(edge_hbm, dis_hbm, xwp_hbm, out_hbm, s_hbm,
              dis_v, xwp0, xwp1, ac0, ac1, ac2, ac3,
              s_v, eiA, eiB, tail_v, semA, semB, semC):
    xwp_cols = (xwp0, xwp1)
    acc_cols = (ac0, ac1, ac2, ac3)
    wid = lax.axis_index("s") * NC + lax.axis_index("c")
    fg = wid % FG
    sh = wid // FG
    ebase = sh * EDGES_SH

    def start(c, buf, sem):
        pltpu.async_copy(
            edge_hbm.at[pl.ds(0, 2), pl.ds(ebase + c * CHUNK, CHUNK)], buf, sem)

    def drain(buf, sem):
        pltpu.make_async_copy(
            edge_hbm.at[pl.ds(0, 2), pl.ds(0, CHUNK)], buf, sem).wait()

    start(0, eiA, semA)

    stage = [pltpu.make_async_copy(dis_hbm, dis_v, semC)]
    for j in range(FPT // 2):
        stage.append(pltpu.make_async_copy(
            xwp_hbm.at[fg * 2 + j], xwp_cols[j], semC))
    for cp in stage:
        cp.start()

    zeros = jnp.zeros((L,), jnp.float32)

    @pl.loop(0, N // L, unroll=8)
    def _zero(i):
        s_v[pl.ds(i * L, L)] = zeros
        for f in range(FPT):
            acc_cols[f][pl.ds(i * L, L)] = zeros

    for cp in stage:
        cp.wait()

    mask_s = jnp.broadcast_to(fg == 0, (L,))

    def process(buf, ngrp=GRP):
        @plsc.parallel_loop(0, ngrp, unroll=4)
        def _grp(i):
            s16 = buf[0, pl.ds(i * L, L)]
            d16 = buf[1, pl.ds(i * L, L)]
            b = plsc.load_gather(dis_v, [d16])
            for j in range(FPT // 2):
                gp = plsc.load_gather(xwp_cols[j], [s16])
                g0, g1 = plsc.unpack(
                    plsc.bitcast(gp, jnp.bfloat16),
                    format=plsc.PackFormat.INTERLEAVED,
                    preferred_element_type=jnp.float32)
                plsc.addupdate_scatter(acc_cols[2 * j], [d16], g0 * b)
                plsc.addupdate_scatter(acc_cols[2 * j + 1], [d16], g1 * b)
            plsc.addupdate_scatter(s_v, [s16], b, mask=mask_s)

    @pl.loop(0, NCHUNK, step=2)
    def _chunk(c):
        drain(eiA, semA)
        start(c + 1, eiB, semB)
        process(eiA)
        drain(eiB, semB)

        @pl.when(c + 2 < NCHUNK)
        def _():
            start(c + 2, eiA, semA)

        process(eiB)

    @pl.when(sh == 0)
    def _tail():
        pltpu.sync_copy(
            edge_hbm.at[pl.ds(0, 2), pl.ds(TAIL_BASE, TAIL)], tail_v)
        process(tail_v, GRP_T)

    for j in range(FPT // 2):
        pltpu.sync_copy(acc_cols[2 * j], out_hbm.at[sh, fg * 2 + j])
        pltpu.sync_copy(acc_cols[2 * j + 1], out_hbm.at[sh, fg * 2 + j + H // 2])
    pltpu.sync_copy(s_v, s_hbm.at[wid])


def _tc_finish_body(outp_ref, sp_ref, dis_ref, xw2t_ref, b1_ref, w2_ref,
                    b2_ref, o_ref):
    dis = dis_ref[...]
    edge = jnp.sum(outp_ref[...], axis=0)
    h = edge + dis * xw2t_ref[...] + b1_ref[...]
    h = jnp.maximum(h, 0.0)
    s = jnp.sum(sp_ref[...], axis=0, keepdims=True)
    cvec = dis * (s + dis)
    v = jnp.sum(h * cvec, axis=1, keepdims=True)
    pooled = jnp.sum(v * w2_ref[...], axis=0, keepdims=True) / N + b2_ref[...]
    m = jnp.max(pooled)
    e = jnp.exp(pooled - m)
    o_ref[...] = e / jnp.sum(e)


_tc_finish = pl.pallas_call(
    _tc_finish_body,
    out_shape=jax.ShapeDtypeStruct((1, C), jnp.float32),
)


def kernel(x, edge_index, W1, b1, W2, b2):
    edge_index = edge_index.astype(jnp.int32)

    xw = _tc_matmul(x, W1)
    deg_part = _sc_degree(edge_index)
    dis2d, xw2t, xwp = _tc_scale(deg_part, xw)
    dis = dis2d.reshape(N)

    out_part, s_part = _sc_edges(edge_index, dis, xwp)
    out = _tc_finish(out_part, s_part, dis2d, xw2t,
                     b1.reshape(H, 1), W2, b2.reshape(1, C))
    return out
